# baseline (speedup 1.0000x reference)
import functools

import jax
import jax.numpy as jnp
from jax import lax
from jax.experimental import pallas as pl
from jax.experimental.pallas import tpu as pltpu
from jax.experimental.pallas import tpu_sc as plsc

_MARGIN_P = 0.5
_MARGIN_N = 1.5
_EPS = 1e-09

_N = 4096
_D = 128
_ROWS_PER_BLOCK = 128
_GRID = _N // _ROWS_PER_BLOCK


def _tc_body(o1_ref, o2_ref, tgt_ref, nv_ref, sump_ref):
    i = pl.program_id(0)

    @pl.when(i == 0)
    def _init():
        sump_ref[...] = jnp.zeros((1, 128), jnp.float32)

    diff = o2_ref[...] - o1_ref[...]
    d = jnp.sum(diff * diff, axis=1, keepdims=True)
    s = jnp.sqrt(d + _EPS)
    loss_p = 0.5 * jnp.maximum(s - _MARGIN_P, 0.0) ** 2
    loss_n = 0.5 * jnp.maximum(_MARGIN_N - s, 0.0) ** 2
    mask = tgt_ref[...] != 0
    nv_ref[...] = jnp.where(mask, jnp.float32(-1.0), loss_n)
    partial_p = jnp.sum(jnp.where(mask, loss_p, 0.0))
    sump_ref[...] += partial_p


def _tc_losses(o1, o2, tgt2d):
    return pl.pallas_call(
        _tc_body,
        grid=(_GRID,),
        in_specs=[
            pl.BlockSpec((_ROWS_PER_BLOCK, _D), lambda i: (i, 0)),
            pl.BlockSpec((_ROWS_PER_BLOCK, _D), lambda i: (i, 0)),
            pl.BlockSpec((_ROWS_PER_BLOCK, 1), lambda i: (i, 0)),
        ],
        out_specs=[
            pl.BlockSpec((_ROWS_PER_BLOCK, 1), lambda i: (i, 0)),
            pl.BlockSpec((1, 128), lambda i: (0, 0)),
        ],
        out_shape=[
            jax.ShapeDtypeStruct((_N, 1), jnp.float32),
            jax.ShapeDtypeStruct((1, 128), jnp.float32),
        ],
    )(o1, o2, tgt2d)


_CHUNKS = _N // 16
_OUTER = 16


def _sc_body(nv_hbm, sump_hbm, out_hbm, nv_v, sump_v, out_v):
    c = lax.axis_index("c")
    s = lax.axis_index("s")

    @pl.when(jnp.logical_and(c == 0, s == 0))
    def _():
        pltpu.sync_copy(nv_hbm, nv_v)
        pltpu.sync_copy(sump_hbm, sump_v)

        def p1(i, carry):
            mx, cnt = carry
            for j in range(16):
                v = nv_v[pl.ds(i * 256 + j * 16, 16)]
                mx = jnp.maximum(mx, v)
                cnt = cnt + jnp.where(v >= 0.0, 1, 0).astype(jnp.int32)
            return mx, cnt

        mx0 = jnp.full((16,), -1.0, jnp.float32)
        cnt0 = jnp.zeros((16,), jnp.int32)
        mx, cnt = lax.fori_loop(0, _OUTER, p1, (mx0, cnt0))
        maxf = jnp.max(mx)
        n_neg = jnp.sum(cnt)
        num_pos = _N - n_neg
        k = jnp.minimum(jnp.maximum(1, num_pos), n_neg)

        hi0 = lax.bitcast_convert_type(maxf, jnp.int32) + 1
        lo0 = jnp.int32(0)

        def count_ge(t):
            def body(i, acc):
                for j in range(16):
                    v = nv_v[pl.ds(i * 256 + j * 16, 16)]
                    acc = acc + jnp.where(v >= t, 1, 0).astype(jnp.int32)
                return acc

            acc = lax.fori_loop(0, _OUTER, body, jnp.zeros((16,), jnp.int32))
            return jnp.sum(acc)

        def w_cond(st):
            lo, hi = st
            return (hi - lo) > 1

        def w_body(st):
            lo, hi = st
            mid = lo + ((hi - lo) >> 1)
            t = lax.bitcast_convert_type(mid, jnp.float32)
            ge = count_ge(t)
            take = ge >= k
            return jnp.where(take, mid, lo), jnp.where(take, hi, mid)

        t_bits, _ = lax.while_loop(w_cond, w_body, (lo0, hi0))
        t = lax.bitcast_convert_type(t_bits, jnp.float32)

        def p3(i, carry):
            sa, ga = carry
            for j in range(16):
                v = nv_v[pl.ds(i * 256 + j * 16, 16)]
                gt = v > t
                sa = sa + jnp.where(gt, v, 0.0)
                ga = ga + jnp.where(gt, 1, 0).astype(jnp.int32)
            return sa, ga

        sa, ga = lax.fori_loop(
            0, _OUTER, p3,
            (jnp.zeros((16,), jnp.float32), jnp.zeros((16,), jnp.int32)),
        )
        sum_gt = jnp.sum(sa)
        cnt_gt = jnp.sum(ga)
        sum_n = sum_gt + t * (k - cnt_gt).astype(jnp.float32)
        sum_n = jnp.where(n_neg > 0, sum_n, 0.0)

        sum_p = sump_v[pl.ds(0, 16)][0]
        num = jnp.full((16,), sum_p + sum_n, jnp.float32)
        den = jnp.full((16,), (num_pos + k).astype(jnp.float32), jnp.float32)
        out_v[...] = num / den
        pltpu.sync_copy(out_v, out_hbm)


@functools.partial(jax.jit, donate_argnums=())
def _run(output1, output2, target):
    tgt2d = target.reshape(_N, 1)
    nv, sump = _tc_losses(output1, output2, tgt2d)
    sc = pl.kernel(
        _sc_body,
        out_type=jax.ShapeDtypeStruct((16,), jnp.float32),
        mesh=plsc.VectorSubcoreMesh(core_axis_name="c", subcore_axis_name="s"),
        compiler_params=pltpu.CompilerParams(needs_layout_passes=False),
        scratch_types=[
            pltpu.VMEM((_N,), jnp.float32),
            pltpu.VMEM((128,), jnp.float32),
            pltpu.VMEM((16,), jnp.float32),
        ],
    )
    out16 = sc(nv.reshape(_N), sump.reshape(128))
    return out16[0]


def kernel(output1, output2, target):
    return _run(output1, output2, target)


# speedup vs baseline: 2.1757x; 2.1757x over previous
import functools

import jax
import jax.numpy as jnp
from jax import lax
from jax.experimental import pallas as pl
from jax.experimental.pallas import tpu as pltpu

_MARGIN_P = 0.5
_MARGIN_N = 1.5
_EPS = 1e-09

_N = 4096
_D = 128
_ROWS = 128
_GRID = _N // _ROWS


def _body(o1_ref, o2_ref, tgt_ref, out_ref, nv_s, accp_s, accnp_s):
    i = pl.program_id(0)

    @pl.when(i == 0)
    def _init():
        accp_s[...] = jnp.zeros((1, 128), jnp.float32)
        accnp_s[...] = jnp.zeros((1, 128), jnp.int32)

    diff = o2_ref[...] - o1_ref[...]
    d = jnp.sum(diff * diff, axis=1).reshape(1, 128)
    s = jnp.sqrt(d + _EPS)
    loss_p = 0.5 * jnp.maximum(s - _MARGIN_P, 0.0) ** 2
    loss_n = 0.5 * jnp.maximum(_MARGIN_N - s, 0.0) ** 2
    tgt = tgt_ref[0]
    mask = tgt != 0
    nv_s[pl.ds(i, 1), :] = jnp.where(mask, jnp.float32(-1.0), loss_n)
    accp_s[...] += jnp.where(mask, loss_p, 0.0)
    accnp_s[...] += tgt

    @pl.when(i == _GRID - 1)
    def _finish():
        nv = nv_s[...]
        num_pos = jnp.sum(accnp_s[...])
        n_neg = _N - num_pos
        k = jnp.minimum(jnp.maximum(1, num_pos), n_neg)

        maxv = jnp.max(nv)
        hi0 = lax.bitcast_convert_type(maxv, jnp.int32) + 1
        lo0 = jnp.int32(0)

        def w_cond(st):
            lo, hi = st
            return (hi - lo) > 1

        def w_body(st):
            lo, hi = st
            mid = lo + ((hi - lo) >> 1)
            t = lax.bitcast_convert_type(mid, jnp.float32)
            ge = jnp.sum((nv >= t).astype(jnp.int32))
            take = ge >= k
            return jnp.where(take, mid, lo), jnp.where(take, hi, mid)

        t_bits, _ = lax.while_loop(w_cond, w_body, (lo0, hi0))
        t = lax.bitcast_convert_type(t_bits, jnp.float32)

        gt = nv > t
        sum_gt = jnp.sum(jnp.where(gt, nv, 0.0))
        cnt_gt = jnp.sum(gt.astype(jnp.int32))
        sum_n = sum_gt + t * (k - cnt_gt).astype(jnp.float32)
        sum_n = jnp.where(n_neg > 0, sum_n, 0.0)

        sum_p = jnp.sum(accp_s[...])
        total = (sum_p + sum_n) / (num_pos + k).astype(jnp.float32)
        out_ref[...] = jnp.full((1, 1), total, jnp.float32)


@jax.jit
def _run(output1, output2, target):
    tgt3d = target.reshape(_GRID, 1, _ROWS)
    out = pl.pallas_call(
        _body,
        grid=(_GRID,),
        in_specs=[
            pl.BlockSpec((_ROWS, _D), lambda i: (i, 0)),
            pl.BlockSpec((_ROWS, _D), lambda i: (i, 0)),
            pl.BlockSpec((1, 1, _ROWS), lambda i: (i, 0, 0)),
        ],
        out_specs=pl.BlockSpec((1, 1), lambda i: (0, 0)),
        out_shape=jax.ShapeDtypeStruct((1, 1), jnp.float32),
        scratch_shapes=[
            pltpu.VMEM((_GRID, 128), jnp.float32),
            pltpu.VMEM((1, 128), jnp.float32),
            pltpu.VMEM((1, 128), jnp.int32),
        ],
    )(output1, output2, tgt3d)
    return out[0, 0]


def kernel(output1, output2, target):
    return _run(output1, output2, target)


# speedup vs baseline: 6.9061x; 3.1743x over previous
import jax
import jax.numpy as jnp
from jax import lax
from jax.experimental import pallas as pl
from jax.experimental.pallas import tpu as pltpu

_MARGIN_P = 0.5
_MARGIN_N = 1.5
_EPS = 1e-09

_N = 4096
_D = 128
_SUB = 4
_ROWS = 128 * _SUB
_GRID = _N // _ROWS


def _body(o1_ref, o2_ref, tgt_ref, out_ref, nv_s, accp_s, accnp_s):
    i = pl.program_id(0)

    @pl.when(i == 0)
    def _init():
        accp_s[...] = jnp.zeros((1, 128), jnp.float32)
        accnp_s[...] = jnp.zeros((1, 128), jnp.int32)

    ones_c = jnp.ones((_D, 1), jnp.float32)
    tgt = tgt_ref[0]
    accp = accp_s[...]
    accnp = accnp_s[...]
    for j in range(_SUB):
        diff = o2_ref[pl.ds(j * 128, 128), :] - o1_ref[pl.ds(j * 128, 128), :]
        sq = diff * diff
        d = lax.dot_general(
            ones_c, sq, (((0,), (1,)), ((), ())),
            preferred_element_type=jnp.float32,
        )
        s = jnp.sqrt(d + _EPS)
        loss_p = 0.5 * jnp.maximum(s - _MARGIN_P, 0.0) ** 2
        loss_n = 0.5 * jnp.maximum(_MARGIN_N - s, 0.0) ** 2
        tgt_j = tgt[:, j * 128:(j + 1) * 128]
        mask = tgt_j != 0
        nv_s[pl.ds(i * _SUB + j, 1), :] = jnp.where(
            mask, jnp.float32(-1.0), loss_n)
        accp = accp + jnp.where(mask, loss_p, 0.0)
        accnp = accnp + tgt_j
    accp_s[...] = accp
    accnp_s[...] = accnp

    @pl.when(i == _GRID - 1)
    def _finish():
        nv = nv_s[...]
        num_pos = jnp.sum(accnp_s[...])
        n_neg = _N - num_pos
        k = jnp.minimum(jnp.maximum(1, num_pos), n_neg)

        maxv = jnp.max(nv)
        hi0 = lax.bitcast_convert_type(maxv, jnp.int32) + 1
        lo0 = jnp.int32(0)

        def w_cond(st):
            lo, hi = st
            return (hi - lo) > 1

        def w_body(st):
            lo, hi = st
            mid = lo + ((hi - lo) >> 1)
            t = lax.bitcast_convert_type(mid, jnp.float32)
            ge = jnp.sum((nv >= t).astype(jnp.int32))
            take = ge >= k
            return jnp.where(take, mid, lo), jnp.where(take, hi, mid)

        t_bits, _ = lax.while_loop(w_cond, w_body, (lo0, hi0))
        t = lax.bitcast_convert_type(t_bits, jnp.float32)

        gt = nv > t
        sum_gt = jnp.sum(jnp.where(gt, nv, 0.0))
        cnt_gt = jnp.sum(gt.astype(jnp.int32))
        sum_n = sum_gt + t * (k - cnt_gt).astype(jnp.float32)
        sum_n = jnp.where(n_neg > 0, sum_n, 0.0)

        sum_p = jnp.sum(accp_s[...])
        total = (sum_p + sum_n) / (num_pos + k).astype(jnp.float32)
        out_ref[...] = jnp.full((1, 1), total, jnp.float32)


@jax.jit
def _run(output1, output2, target):
    tgt3d = target.reshape(_GRID, 1, _SUB * 128)
    out = pl.pallas_call(
        _body,
        grid=(_GRID,),
        in_specs=[
            pl.BlockSpec((_ROWS, _D), lambda i: (i, 0)),
            pl.BlockSpec((_ROWS, _D), lambda i: (i, 0)),
            pl.BlockSpec((1, 1, _SUB * 128), lambda i: (i, 0, 0)),
        ],
        out_specs=pl.BlockSpec((1, 1), lambda i: (0, 0)),
        out_shape=jax.ShapeDtypeStruct((1, 1), jnp.float32),
        scratch_shapes=[
            pltpu.VMEM((_N // 128, 128), jnp.float32),
            pltpu.VMEM((1, 128), jnp.float32),
            pltpu.VMEM((1, 128), jnp.int32),
        ],
    )(output1, output2, tgt3d)
    return out[0, 0]


def kernel(output1, output2, target):
    return _run(output1, output2, target)


# speedup vs baseline: 10.0728x; 1.4585x over previous
import jax
import jax.numpy as jnp
from jax import lax
from jax.experimental import pallas as pl
from jax.experimental.pallas import tpu as pltpu

_MARGIN_P = 0.5
_MARGIN_N = 1.5
_EPS = 1e-09

_N = 4096
_D = 128
_SUB = 8
_ROWS = 128 * _SUB
_GRID = _N // _ROWS


def _body(o1_ref, o2_ref, tgt_ref, out_ref, nv_s, accp_s, accnp_s):
    i = pl.program_id(0)

    @pl.when(i == 0)
    def _init():
        accp_s[...] = jnp.zeros((1, 128), jnp.float32)
        accnp_s[...] = jnp.zeros((1, 128), jnp.int32)

    ones_c = jnp.ones((_D, 1), jnp.float32)
    tgt = tgt_ref[0]
    accp = accp_s[...]
    accnp = accnp_s[...]
    for j in range(_SUB):
        diff = o2_ref[pl.ds(j * 128, 128), :] - o1_ref[pl.ds(j * 128, 128), :]
        sq = diff * diff
        d = lax.dot_general(
            ones_c, sq, (((0,), (1,)), ((), ())),
            preferred_element_type=jnp.float32,
        )
        s = jnp.sqrt(d + _EPS)
        loss_p = 0.5 * jnp.maximum(s - _MARGIN_P, 0.0) ** 2
        loss_n = 0.5 * jnp.maximum(_MARGIN_N - s, 0.0) ** 2
        tgt_j = tgt[:, j * 128:(j + 1) * 128]
        mask = tgt_j != 0
        nv_s[pl.ds(i * _SUB + j, 1), :] = jnp.where(
            mask, jnp.float32(-1.0), loss_n)
        accp = accp + jnp.where(mask, loss_p, 0.0)
        accnp = accnp + tgt_j
    accp_s[...] = accp
    accnp_s[...] = accnp

    @pl.when(i == _GRID - 1)
    def _finish():
        nv = nv_s[...]
        num_pos = jnp.sum(accnp_s[...])
        n_neg = _N - num_pos
        k = jnp.minimum(jnp.maximum(1, num_pos), n_neg)

        maxv = jnp.max(nv)
        hi0 = lax.bitcast_convert_type(maxv, jnp.int32) + 1
        lo0 = jnp.int32(0)

        def w_cond(st):
            lo, hi = st
            return (hi - lo) > 1

        def w_body(st):
            lo, hi = st
            mid = lo + ((hi - lo) >> 1)
            t = lax.bitcast_convert_type(mid, jnp.float32)
            ge = jnp.sum((nv >= t).astype(jnp.int32))
            take = ge >= k
            return jnp.where(take, mid, lo), jnp.where(take, hi, mid)

        t_bits, _ = lax.while_loop(w_cond, w_body, (lo0, hi0))
        t = lax.bitcast_convert_type(t_bits, jnp.float32)

        gt = nv > t
        sum_gt = jnp.sum(jnp.where(gt, nv, 0.0))
        cnt_gt = jnp.sum(gt.astype(jnp.int32))
        sum_n = sum_gt + t * (k - cnt_gt).astype(jnp.float32)
        sum_n = jnp.where(n_neg > 0, sum_n, 0.0)

        sum_p = jnp.sum(accp_s[...])
        total = (sum_p + sum_n) / (num_pos + k).astype(jnp.float32)
        out_ref[...] = jnp.full((1, 1), total, jnp.float32)


@jax.jit
def _run(output1, output2, target):
    tgt3d = target.reshape(_GRID, 1, _SUB * 128)
    out = pl.pallas_call(
        _body,
        grid=(_GRID,),
        in_specs=[
            pl.BlockSpec((_ROWS, _D), lambda i: (i, 0)),
            pl.BlockSpec((_ROWS, _D), lambda i: (i, 0)),
            pl.BlockSpec((1, 1, _SUB * 128), lambda i: (i, 0, 0)),
        ],
        out_specs=pl.BlockSpec((1, 1), lambda i: (0, 0)),
        out_shape=jax.ShapeDtypeStruct((1, 1), jnp.float32),
        scratch_shapes=[
            pltpu.VMEM((_N // 128, 128), jnp.float32),
            pltpu.VMEM((1, 128), jnp.float32),
            pltpu.VMEM((1, 128), jnp.int32),
        ],
    )(output1, output2, tgt3d)
    return out[0, 0]


def kernel(output1, output2, target):
    return _run(output1, output2, target)


# speedup vs baseline: 12.9868x; 1.2893x over previous
import jax
import jax.numpy as jnp
from jax import lax
from jax.experimental import pallas as pl
from jax.experimental.pallas import tpu as pltpu

_MARGIN_P = 0.5
_MARGIN_N = 1.5
_EPS = 1e-09

_N = 4096
_D = 128
_SUB = 16
_ROWS = 128 * _SUB
_GRID = _N // _ROWS


def _body(o1_ref, o2_ref, tgt_ref, out_ref, nv_s, accp_s, accnp_s):
    i = pl.program_id(0)

    @pl.when(i == 0)
    def _init():
        accp_s[...] = jnp.zeros((1, 128), jnp.float32)
        accnp_s[...] = jnp.zeros((1, 128), jnp.int32)

    ones_c = jnp.ones((_D, 1), jnp.float32)
    tgt = tgt_ref[0]
    accp = accp_s[...]
    accnp = accnp_s[...]
    for j in range(_SUB):
        diff = o2_ref[pl.ds(j * 128, 128), :] - o1_ref[pl.ds(j * 128, 128), :]
        sq = diff * diff
        d = lax.dot_general(
            ones_c, sq, (((0,), (1,)), ((), ())),
            preferred_element_type=jnp.float32,
        )
        s = jnp.sqrt(d + _EPS)
        loss_p = 0.5 * jnp.maximum(s - _MARGIN_P, 0.0) ** 2
        loss_n = 0.5 * jnp.maximum(_MARGIN_N - s, 0.0) ** 2
        tgt_j = tgt[:, j * 128:(j + 1) * 128]
        mask = tgt_j != 0
        nv_s[pl.ds(i * _SUB + j, 1), :] = jnp.where(
            mask, jnp.float32(-1.0), loss_n)
        accp = accp + jnp.where(mask, loss_p, 0.0)
        accnp = accnp + tgt_j
    accp_s[...] = accp
    accnp_s[...] = accnp

    @pl.when(i == _GRID - 1)
    def _finish():
        nv = nv_s[...]
        num_pos = jnp.sum(accnp_s[...])
        n_neg = _N - num_pos
        k = jnp.minimum(jnp.maximum(1, num_pos), n_neg)

        maxv = jnp.max(nv)
        hi0 = lax.bitcast_convert_type(maxv, jnp.int32) + 1
        lo0 = jnp.int32(0)

        def w_cond(st):
            lo, hi = st
            return (hi - lo) > 1

        def w_body(st):
            lo, hi = st
            mid = lo + ((hi - lo) >> 1)
            t = lax.bitcast_convert_type(mid, jnp.float32)
            ge = jnp.sum((nv >= t).astype(jnp.int32))
            take = ge >= k
            return jnp.where(take, mid, lo), jnp.where(take, hi, mid)

        t_bits, _ = lax.while_loop(w_cond, w_body, (lo0, hi0))
        t = lax.bitcast_convert_type(t_bits, jnp.float32)

        gt = nv > t
        sum_gt = jnp.sum(jnp.where(gt, nv, 0.0))
        cnt_gt = jnp.sum(gt.astype(jnp.int32))
        sum_n = sum_gt + t * (k - cnt_gt).astype(jnp.float32)
        sum_n = jnp.where(n_neg > 0, sum_n, 0.0)

        sum_p = jnp.sum(accp_s[...])
        total = (sum_p + sum_n) / (num_pos + k).astype(jnp.float32)
        out_ref[...] = jnp.full((1, 1), total, jnp.float32)


@jax.jit
def _run(output1, output2, target):
    tgt3d = target.reshape(_GRID, 1, _SUB * 128)
    out = pl.pallas_call(
        _body,
        grid=(_GRID,),
        in_specs=[
            pl.BlockSpec((_ROWS, _D), lambda i: (i, 0)),
            pl.BlockSpec((_ROWS, _D), lambda i: (i, 0)),
            pl.BlockSpec((1, 1, _SUB * 128), lambda i: (i, 0, 0)),
        ],
        out_specs=pl.BlockSpec((1, 1), lambda i: (0, 0)),
        out_shape=jax.ShapeDtypeStruct((1, 1), jnp.float32),
        scratch_shapes=[
            pltpu.VMEM((_N // 128, 128), jnp.float32),
            pltpu.VMEM((1, 128), jnp.float32),
            pltpu.VMEM((1, 128), jnp.int32),
        ],
    )(output1, output2, tgt3d)
    return out[0, 0]


def kernel(output1, output2, target):
    return _run(output1, output2, target)
